# baseline (speedup 1.0000x reference)
import functools

import jax
import jax.numpy as jnp
from jax.experimental import pallas as pl
from jax.experimental.pallas import tpu as pltpu

B, N, D = 256, 128, 128
H, QD, KD, VD = 4, 32, 32, 32


def _leaky(x, alpha=0.2):
    return jnp.where(x >= 0, x, alpha * x)


def _egatv2_kernel(e_ref, x_ref, m_ref, wq_ref, wk_ref, wv_ref,
                   aq_ref, ak_ref, eps_ref, out_ref):
    x = x_ref[0]
    q_all = _leaky(jnp.dot(x, wq_ref[...], preferred_element_type=jnp.float32))
    k_all = _leaky(jnp.dot(x, wk_ref[...], preferred_element_type=jnp.float32))
    v_all = jnp.dot(x, wv_ref[...], preferred_element_type=jnp.float32)

    eps = eps_ref[0, 0]
    row = jax.lax.broadcasted_iota(jnp.int32, (N, N), 0)
    col = jax.lax.broadcasted_iota(jnp.int32, (N, N), 1)
    diag = row == col
    base = e_ref[0] + jnp.where(diag, eps, jnp.float32(0.0))
    keep = (m_ref[0] > 0.5) | diag

    for h in range(H):
        qh = q_all[:, h * QD:(h + 1) * QD]
        kh = k_all[:, h * KD:(h + 1) * KD]
        sq = jax.lax.dot_general(qh, aq_ref[h:h + 1, :],
                                 (((1,), (1,)), ((), ())),
                                 preferred_element_type=jnp.float32)
        sk = jax.lax.dot_general(ak_ref[h:h + 1, :], kh,
                                 (((1,), (1,)), ((), ())),
                                 preferred_element_type=jnp.float32)
        logits = jnp.where(keep, base + sq + sk, jnp.float32(-1e9))
        logits = logits - jnp.max(logits, axis=1, keepdims=True)
        p = jnp.exp(logits)
        attn = p / jnp.sum(p, axis=1, keepdims=True)
        out_ref[0, :, h * VD:(h + 1) * VD] = jnp.dot(
            attn, v_all[:, h * VD:(h + 1) * VD],
            preferred_element_type=jnp.float32)


@functools.partial(jax.jit, static_argnames=("interpret",))
def kernel(e, x_atm, m, Wq, Wk, Wv, aq, ak, eps, interpret=False):
    wq_flat = jnp.transpose(Wq, (1, 0, 2)).reshape(D, H * QD)
    wk_flat = jnp.transpose(Wk, (1, 0, 2)).reshape(D, H * KD)
    wv_flat = jnp.transpose(Wv, (1, 0, 2)).reshape(D, H * VD)
    eps2 = eps.reshape(1, 1)

    full = lambda shape: pl.BlockSpec(shape, lambda b: (0,) * len(shape))
    per_b = lambda shape: pl.BlockSpec(shape, lambda b: (b, 0, 0))

    return pl.pallas_call(
        _egatv2_kernel,
        grid=(B,),
        in_specs=[
            per_b((1, N, N)),
            per_b((1, N, D)),
            per_b((1, N, N)),
            full((D, H * QD)),
            full((D, H * KD)),
            full((D, H * VD)),
            full((H, QD)),
            full((H, KD)),
            pl.BlockSpec(memory_space=pltpu.SMEM),
        ],
        out_specs=per_b((1, N, H * VD)),
        out_shape=jax.ShapeDtypeStruct((B, N, H * VD), jnp.float32),
        compiler_params=pltpu.CompilerParams(
            dimension_semantics=("parallel",)),
        interpret=interpret,
    )(e, x_atm, m, wq_flat, wk_flat, wv_flat, aq, ak, eps2)


# speedup vs baseline: 6.9345x; 6.9345x over previous
import functools

import jax
import jax.numpy as jnp
from jax.experimental import pallas as pl
from jax.experimental.pallas import tpu as pltpu

B, N, D = 256, 128, 128
H, QD, KD, VD = 4, 32, 32, 32
G = 8


def _leaky(x, alpha=0.2):
    return jnp.where(x >= 0, x, alpha * x)


def _egatv2_kernel(e_ref, x_ref, m_ref, wk_ref, wv_ref, akf_ref, sel_ref,
                   selt_ref, eye_ref, epseye_ref, out_ref):
    x2 = x_ref[...].reshape(G * N, D)
    k_all = _leaky(jnp.dot(x2, wk_ref[...], preferred_element_type=jnp.float32))
    v_all = jnp.dot(x2, wv_ref[...], preferred_element_type=jnp.float32)

    skn = jnp.dot(k_all * akf_ref[...], sel_ref[...],
                  preferred_element_type=jnp.float32)
    esk = jnp.exp(skn)
    et = jnp.dot(esk, selt_ref[...], preferred_element_type=jnp.float32)
    vsc = v_all * et

    eye = eye_ref[...]
    epseye = epseye_ref[...]
    for g in range(G):
        expbase = (jnp.exp(e_ref[g] + epseye)
                   * jnp.maximum(m_ref[g], eye))
        sl = slice(g * N, (g + 1) * N)
        denom = jnp.dot(expbase, esk[sl, :],
                        preferred_element_type=jnp.float32)
        rt = jnp.dot(1.0 / denom, selt_ref[...],
                     preferred_element_type=jnp.float32)
        o = jnp.dot(expbase, vsc[sl, :], preferred_element_type=jnp.float32)
        out_ref[g] = o * rt


@functools.partial(jax.jit, static_argnames=("interpret",))
def kernel(e, x_atm, m, Wq, Wk, Wv, aq, ak, eps, interpret=False):
    del Wq, aq
    wk_flat = jnp.transpose(Wk, (1, 0, 2)).reshape(D, H * KD)
    wv_flat = jnp.transpose(Wv, (1, 0, 2)).reshape(D, H * VD)
    ak_flat = ak.reshape(1, H * KD)
    sel = (jnp.arange(H * KD)[:, None] // KD ==
           jnp.arange(H)[None, :]).astype(jnp.float32)
    eye = jnp.eye(N, dtype=jnp.float32)
    epseye = eps[0] * eye

    full = lambda shape: pl.BlockSpec(shape, lambda b: (0,) * len(shape))
    per_b = lambda shape: pl.BlockSpec(shape, lambda b: (b, 0, 0))

    return pl.pallas_call(
        _egatv2_kernel,
        grid=(B // G,),
        in_specs=[
            per_b((G, N, N)),
            per_b((G, N, D)),
            per_b((G, N, N)),
            full((D, H * KD)),
            full((D, H * VD)),
            full((1, H * KD)),
            full((H * KD, H)),
            full((H, H * VD)),
            full((N, N)),
            full((N, N)),
        ],
        out_specs=per_b((G, N, H * VD)),
        out_shape=jax.ShapeDtypeStruct((B, N, H * VD), jnp.float32),
        compiler_params=pltpu.CompilerParams(
            dimension_semantics=("parallel",)),
        interpret=interpret,
    )(e, x_atm, m, wk_flat, wv_flat, ak_flat, sel, sel.T, eye, epseye)


# speedup vs baseline: 7.4361x; 1.0723x over previous
import functools

import jax
import jax.numpy as jnp
from jax.experimental import pallas as pl
from jax.experimental.pallas import tpu as pltpu

B, N, D = 256, 128, 128
H, QD, KD, VD = 4, 32, 32, 32
G = 16


def _leaky(x, alpha=0.2):
    return jnp.where(x >= 0, x, alpha * x)


def _egatv2_kernel(e_ref, x_ref, m_ref, wk_ref, wv_ref, akf_ref, sel_ref,
                   selt_ref, eye_ref, epseye_ref, out_ref):
    x2 = x_ref[...].reshape(G * N, D)
    k_all = _leaky(jnp.dot(x2, wk_ref[...], preferred_element_type=jnp.float32))
    v_all = jnp.dot(x2, wv_ref[...], preferred_element_type=jnp.float32)

    skn = jnp.dot(k_all * akf_ref[...], sel_ref[...],
                  preferred_element_type=jnp.float32)
    esk = jnp.exp(skn)
    et = jnp.dot(esk, selt_ref[...], preferred_element_type=jnp.float32)
    vsc = v_all * et

    eye = eye_ref[...]
    epseye = epseye_ref[...]
    for g in range(G):
        expbase = (jnp.exp(e_ref[g] + epseye)
                   * jnp.maximum(m_ref[g], eye))
        sl = slice(g * N, (g + 1) * N)
        denom = jnp.dot(expbase, esk[sl, :],
                        preferred_element_type=jnp.float32)
        rt = jnp.dot(1.0 / denom, selt_ref[...],
                     preferred_element_type=jnp.float32)
        o = jnp.dot(expbase, vsc[sl, :], preferred_element_type=jnp.float32)
        out_ref[g] = o * rt


@functools.partial(jax.jit, static_argnames=("interpret",))
def kernel(e, x_atm, m, Wq, Wk, Wv, aq, ak, eps, interpret=False):
    del Wq, aq
    wk_flat = jnp.transpose(Wk, (1, 0, 2)).reshape(D, H * KD)
    wv_flat = jnp.transpose(Wv, (1, 0, 2)).reshape(D, H * VD)
    ak_flat = ak.reshape(1, H * KD)
    sel = (jnp.arange(H * KD)[:, None] // KD ==
           jnp.arange(H)[None, :]).astype(jnp.float32)
    eye = jnp.eye(N, dtype=jnp.float32)
    epseye = eps[0] * eye

    full = lambda shape: pl.BlockSpec(shape, lambda b: (0,) * len(shape))
    per_b = lambda shape: pl.BlockSpec(shape, lambda b: (b, 0, 0))

    return pl.pallas_call(
        _egatv2_kernel,
        grid=(B // G,),
        in_specs=[
            per_b((G, N, N)),
            per_b((G, N, D)),
            per_b((G, N, N)),
            full((D, H * KD)),
            full((D, H * VD)),
            full((1, H * KD)),
            full((H * KD, H)),
            full((H, H * VD)),
            full((N, N)),
            full((N, N)),
        ],
        out_specs=per_b((G, N, H * VD)),
        out_shape=jax.ShapeDtypeStruct((B, N, H * VD), jnp.float32),
        compiler_params=pltpu.CompilerParams(
            dimension_semantics=("parallel",)),
        interpret=interpret,
    )(e, x_atm, m, wk_flat, wv_flat, ak_flat, sel, sel.T, eye, epseye)


# speedup vs baseline: 7.5302x; 1.0127x over previous
import functools

import jax
import jax.numpy as jnp
from jax.experimental import pallas as pl
from jax.experimental.pallas import tpu as pltpu

B, N, D = 256, 128, 128
H, QD, KD, VD = 4, 32, 32, 32
G = 32


def _leaky(x, alpha=0.2):
    return jnp.where(x >= 0, x, alpha * x)


def _egatv2_kernel(e_ref, x_ref, m_ref, wk_ref, wv_ref, akf_ref, sel_ref,
                   selt_ref, eye_ref, epseye_ref, out_ref):
    x2 = x_ref[...].reshape(G * N, D)
    k_all = _leaky(jnp.dot(x2, wk_ref[...], preferred_element_type=jnp.float32))
    v_all = jnp.dot(x2, wv_ref[...], preferred_element_type=jnp.float32)

    skn = jnp.dot(k_all * akf_ref[...], sel_ref[...],
                  preferred_element_type=jnp.float32)
    esk = jnp.exp(skn)
    et = jnp.dot(esk, selt_ref[...], preferred_element_type=jnp.float32)
    vsc = v_all * et

    eye = eye_ref[...]
    epseye = epseye_ref[...]
    for g in range(G):
        expbase = (jnp.exp(e_ref[g] + epseye)
                   * jnp.maximum(m_ref[g], eye))
        sl = slice(g * N, (g + 1) * N)
        denom = jnp.dot(expbase, esk[sl, :],
                        preferred_element_type=jnp.float32)
        rt = jnp.dot(1.0 / denom, selt_ref[...],
                     preferred_element_type=jnp.float32)
        o = jnp.dot(expbase, vsc[sl, :], preferred_element_type=jnp.float32)
        out_ref[g] = o * rt


@functools.partial(jax.jit, static_argnames=("interpret",))
def kernel(e, x_atm, m, Wq, Wk, Wv, aq, ak, eps, interpret=False):
    del Wq, aq
    wk_flat = jnp.transpose(Wk, (1, 0, 2)).reshape(D, H * KD)
    wv_flat = jnp.transpose(Wv, (1, 0, 2)).reshape(D, H * VD)
    ak_flat = ak.reshape(1, H * KD)
    sel = (jnp.arange(H * KD)[:, None] // KD ==
           jnp.arange(H)[None, :]).astype(jnp.float32)
    eye = jnp.eye(N, dtype=jnp.float32)
    epseye = eps[0] * eye

    full = lambda shape: pl.BlockSpec(shape, lambda b: (0,) * len(shape))
    per_b = lambda shape: pl.BlockSpec(shape, lambda b: (b, 0, 0))

    return pl.pallas_call(
        _egatv2_kernel,
        grid=(B // G,),
        in_specs=[
            per_b((G, N, N)),
            per_b((G, N, D)),
            per_b((G, N, N)),
            full((D, H * KD)),
            full((D, H * VD)),
            full((1, H * KD)),
            full((H * KD, H)),
            full((H, H * VD)),
            full((N, N)),
            full((N, N)),
        ],
        out_specs=per_b((G, N, H * VD)),
        out_shape=jax.ShapeDtypeStruct((B, N, H * VD), jnp.float32),
        compiler_params=pltpu.CompilerParams(
            dimension_semantics=("parallel",)),
        interpret=interpret,
    )(e, x_atm, m, wk_flat, wv_flat, ak_flat, sel, sel.T, eye, epseye)
